# baseline (speedup 1.0000x reference)
import functools

import jax
import jax.numpy as jnp
from jax import lax
from jax.experimental import pallas as pl
from jax.experimental.pallas import tpu as pltpu
from jax.experimental.pallas import tpu_sc as plsc

T = 2048
D = 1024
DFF = 4096
E = 8
TOPK = 2
P = T * TOPK

TM = 256
NT = (P + E * TM) // TM
M_PAD = NT * TM

NC, NS, LANES = 2, 16, 16
NW = NC * NS

_F32 = jnp.float32
_I32 = jnp.int32


def _cumsum_rows(x):
    n = x.shape[0]
    s = 1
    while s < n:
        x = x + jnp.concatenate(
            [jnp.zeros((s, x.shape[1]), x.dtype), x[:-s, :]], axis=0)
        s *= 2
    return x


def _cumsum_lanes(x):
    n = x.shape[1]
    s = 1
    while s < n:
        x = x + jnp.concatenate(
            [jnp.zeros((x.shape[0], s), x.dtype), x[:, :-s]], axis=1)
        s *= 2
    return x


def _router_body(xf_ref, wr_ref, br_ref, dst_ref, w0_ref, w1_ref, meta_ref):
    xf = xf_ref[...]
    wr = wr_ref[...]
    logits = lax.dot_general(
        xf, wr, (((1,), (0,)), ((), ())),
        preferred_element_type=_F32)
    logits = logits + br_ref[...]
    lane = lax.broadcasted_iota(_I32, (T, 128), 1)
    neg = _F32(-1e30)
    logits = jnp.where(lane < E, logits, neg)

    m0 = jnp.max(logits, axis=1, keepdims=True)
    e0 = jnp.min(jnp.where(logits == m0, lane, 127), axis=1, keepdims=True)
    l2 = jnp.where(lane == e0, neg, logits)
    m1 = jnp.max(l2, axis=1, keepdims=True)
    e1 = jnp.min(jnp.where(l2 == m1, lane, 127), axis=1, keepdims=True)
    r = jnp.exp(m1 - m0)
    w0 = 1.0 / (1.0 + r)
    w1 = 1.0 - w0
    w0_ref[...] = jnp.broadcast_to(w0, (T, 16))
    w1_ref[...] = jnp.broadcast_to(w1, (T, 16))

    onehot = jnp.concatenate([lane == e0, lane == e1], axis=0).astype(_F32)
    csum = _cumsum_rows(onehot)
    pos = jnp.sum((csum - onehot) * onehot, axis=1, keepdims=True)
    counts = csum[P - 1:P, :]
    cap = jnp.ceil(counts / TM) * TM
    poff = _cumsum_lanes(cap) - cap
    dst = jnp.sum(onehot * poff, axis=1, keepdims=True) + pos
    dst_ref[...] = dst.astype(_I32)

    nact = jnp.sum(cap, axis=1, keepdims=True) / TM
    lane_f = lane.astype(_F32)
    tstart = lax.broadcasted_iota(_I32, (32, 128), 0).astype(_F32) * TM
    belongs = (tstart >= poff) & (tstart < poff + cap) & (counts > 0)
    te = jnp.sum(jnp.where(belongs, lane_f[:32, :], 0.0), axis=1,
                 keepdims=True)
    q = jnp.sum(jnp.where(belongs, poff + cap, 0.0), axis=1, keepdims=True)
    b2g = (q >= poff) & (q < poff + cap) & (counts > 0)
    ne_raw = jnp.sum(jnp.where(b2g, lane_f[:32, :], 0.0), axis=1,
                     keepdims=True)
    has_next = jnp.sum(b2g.astype(_F32), axis=1, keepdims=True) > 0
    ne = jnp.where(has_next, ne_raw, te)
    last_e = jnp.max(jnp.where(counts > 0, lane_f[:1, :], -1.0),
                     axis=1, keepdims=True)
    tile_i = lax.broadcasted_iota(_I32, (32, 1), 0).astype(_F32)
    te = jnp.where(tile_i < nact, te, last_e)
    ne = jnp.where(tile_i < nact, ne, last_e)
    meta_ref[...] = jnp.concatenate([nact, te, ne], axis=0).astype(_I32)


def _router_call(xf, wr_p, br_p):
    return pl.pallas_call(
        _router_body,
        out_shape=(
            jax.ShapeDtypeStruct((P, 1), _I32),
            jax.ShapeDtypeStruct((T, 16), _F32),
            jax.ShapeDtypeStruct((T, 16), _F32),
            jax.ShapeDtypeStruct((65, 1), _I32),
        ),
    )(xf, wr_p, br_p)


def _dispatch_call(xf, dst_b):
    mesh = plsc.VectorSubcoreMesh(core_axis_name="c", subcore_axis_name="s")
    ppw = P // NW
    nch = ppw // LANES

    @functools.partial(
        pl.kernel,
        mesh=mesh,
        out_type=jax.ShapeDtypeStruct((M_PAD, D), _F32),
        scratch_types=[
            pltpu.VMEM((nch, LANES), _I32),
            pltpu.VMEM((2, LANES, D), _F32),
            pltpu.SemaphoreType.DMA,
            pltpu.SemaphoreType.DMA,
            pltpu.SemaphoreType.DMA,
            pltpu.SemaphoreType.DMA,
        ],
    )
    def b_kernel(xf_hbm, dst_hbm, xs_hbm, idx_v, rows_v, sl0, sl1, ss0, ss1):
        wid = lax.axis_index("s") * NC + lax.axis_index("c")
        pltpu.sync_copy(dst_hbm.at[wid], idx_v)
        lsem = (sl0, sl1)
        ssem = (ss0, ss1)

        def load_cp(c):
            j0 = wid * ppw + c * LANES
            t0 = lax.rem(j0, T)
            return pltpu.make_async_copy(
                xf_hbm.at[pl.ds(t0, LANES)], rows_v.at[c % 2], lsem[c % 2])

        def scat_cp(c):
            return pltpu.make_async_copy(
                rows_v.at[c % 2], xs_hbm.at[idx_v.at[c]], ssem[c % 2])

        load_cp(0).start()
        for c in range(nch):
            if c >= 1:
                scat_cp(c - 1).wait()
            if c + 1 < nch:
                load_cp(c + 1).start()
            load_cp(c).wait()
            scat_cp(c).start()
        scat_cp(nch - 1).wait()

    return b_kernel(xf, dst_b)


def _ffn_body(meta_ref, xs_ref, b1_ref, b2_ref, w1_hbm, w2_hbm, y_ref,
              w1b, w2b, w1f, w2f, sem1, sem2):
    i = pl.program_id(0)
    nact = meta_ref[0]
    e = meta_ref[1 + i]
    ne = meta_ref[33 + i]
    first = meta_ref[1 + i] != meta_ref[i]

    def w1_copy(eidx):
        return pltpu.make_async_copy(w1_hbm.at[eidx], w1f, sem1)

    def w2_copy(eidx):
        return pltpu.make_async_copy(w2_hbm.at[eidx], w2f, sem2)

    @pl.when(first & (i == 0))
    def _():
        w1_copy(e).start()
        w2_copy(e).start()

    @pl.when(first)
    def _():
        w1_copy(e).wait()
        w1b[...] = w1f[...].astype(jnp.bfloat16)

    @pl.when(i < nact)
    def _():
        x = xs_ref[...].astype(jnp.bfloat16)
        h = lax.dot_general(
            x, w1b[...], (((1,), (0,)), ((), ())),
            preferred_element_type=_F32)
        h = h + b1_ref[0]
        h = 0.5 * h * (1.0 + lax.erf(h * 0.7071067811865476))
        hb = h.astype(jnp.bfloat16)

        @pl.when(first)
        def _():
            w2_copy(e).wait()
            w2b[...] = w2f[...].astype(jnp.bfloat16)

        @pl.when(first & (ne != e))
        def _():
            w1_copy(ne).start()
            w2_copy(ne).start()

        y = lax.dot_general(
            hb, w2b[...], (((1,), (0,)), ((), ())),
            preferred_element_type=_F32)
        y_ref[...] = y + b2_ref[0]


def _ffn_call(meta, xs, b1r, b2r, w1, w2):
    grid_spec = pltpu.PrefetchScalarGridSpec(
        num_scalar_prefetch=1,
        grid=(NT,),
        in_specs=[
            pl.BlockSpec((TM, D), lambda i, m: (i, 0)),
            pl.BlockSpec((1, 1, DFF), lambda i, m: (m[1 + i], 0, 0)),
            pl.BlockSpec((1, 1, D), lambda i, m: (m[1 + i], 0, 0)),
            pl.BlockSpec(memory_space=pl.ANY),
            pl.BlockSpec(memory_space=pl.ANY),
        ],
        out_specs=pl.BlockSpec((TM, D), lambda i, m: (i, 0)),
        scratch_shapes=[
            pltpu.VMEM((D, DFF), jnp.bfloat16),
            pltpu.VMEM((DFF, D), jnp.bfloat16),
            pltpu.VMEM((D, DFF), _F32),
            pltpu.VMEM((DFF, D), _F32),
            pltpu.SemaphoreType.DMA,
            pltpu.SemaphoreType.DMA,
        ],
    )
    return pl.pallas_call(
        _ffn_body,
        grid_spec=grid_spec,
        out_shape=jax.ShapeDtypeStruct((M_PAD, D), _F32),
    )(meta, xs, b1r, b2r, w1, w2)


def _combine_call(y, d0, d1, w0x, w1x):
    mesh = plsc.VectorSubcoreMesh(core_axis_name="c", subcore_axis_name="s")
    tpw = T // NW
    nch = tpw // LANES

    @functools.partial(
        pl.kernel,
        mesh=mesh,
        out_type=jax.ShapeDtypeStruct((T, D), _F32),
        scratch_types=[
            pltpu.VMEM((nch, LANES), _I32),
            pltpu.VMEM((nch, LANES), _I32),
            pltpu.VMEM((tpw, LANES), _F32),
            pltpu.VMEM((tpw, LANES), _F32),
            pltpu.VMEM((2, LANES, D), _F32),
            pltpu.VMEM((2, LANES, D), _F32),
            pltpu.VMEM((2, LANES, D), _F32),
            pltpu.SemaphoreType.DMA,
            pltpu.SemaphoreType.DMA,
            pltpu.SemaphoreType.DMA,
            pltpu.SemaphoreType.DMA,
        ],
    )
    def d_kernel(y_hbm, d0_hbm, d1_hbm, w0_hbm, w1_hbm, out_hbm,
                 i0, i1, w0v, w1v, y0, y1, ov, sg0, sg1, sw0, sw1):
        wid = lax.axis_index("s") * NC + lax.axis_index("c")
        pltpu.sync_copy(d0_hbm.at[wid], i0)
        pltpu.sync_copy(d1_hbm.at[wid], i1)
        pltpu.sync_copy(w0_hbm.at[pl.ds(wid * tpw, tpw)], w0v)
        pltpu.sync_copy(w1_hbm.at[pl.ds(wid * tpw, tpw)], w1v)
        gsem = (sg0, sg1)
        wsem = (sw0, sw1)

        def g0_cp(c):
            return pltpu.make_async_copy(
                y_hbm.at[i0.at[c]], y0.at[c % 2], gsem[c % 2])

        def g1_cp(c):
            return pltpu.make_async_copy(
                y_hbm.at[i1.at[c]], y1.at[c % 2], gsem[c % 2])

        def out_cp(c):
            tbase = wid * tpw + c * LANES
            return pltpu.make_async_copy(
                ov.at[c % 2], out_hbm.at[pl.ds(tbase, LANES)], wsem[c % 2])

        g0_cp(0).start()
        g1_cp(0).start()
        for c in range(nch):
            if c + 1 < nch:
                g0_cp(c + 1).start()
                g1_cp(c + 1).start()
            g0_cp(c).wait()
            g1_cp(c).wait()
            if c >= 2:
                out_cp(c - 2).wait()
            par = c % 2

            def row(rr, rcarry, c=c, par=par):
                a0 = w0v[c * LANES + rr]
                a1 = w1v[c * LANES + rr]
                for cc in range(D // LANES):
                    sl = pl.ds(cc * LANES, LANES)
                    ov[par, rr, sl] = (a0 * y0[par, rr, sl]
                                       + a1 * y1[par, rr, sl])
                return rcarry

            lax.fori_loop(0, LANES, row, 0)
            out_cp(c).start()
        out_cp(nch - 2).wait()
        out_cp(nch - 1).wait()

    return d_kernel(y, d0, d1, w0x, w1x)


def kernel(x, W1, b1, W2, b2, Wr, br):
    bsz, seq, d = x.shape
    xf = x.reshape(T, D)
    wr_p = jnp.pad(Wr, ((0, 0), (0, 128 - E)))
    br_p = jnp.pad(br, (0, 128 - E)).reshape(1, 128)

    dst, w0x, w1x, meta = _router_call(xf, wr_p, br_p)
    dst_flat = dst.reshape(P)
    dst_b = dst_flat.reshape(NW, P // NW // LANES, LANES)

    xs = _dispatch_call(xf, dst_b)

    y = _ffn_call(meta.reshape(65), xs, b1.reshape(E, 1, DFF),
                  b2.reshape(E, 1, D), W1, W2)

    d0 = dst_flat[:T].reshape(NW, T // NW // LANES, LANES)
    d1 = dst_flat[T:].reshape(NW, T // NW // LANES, LANES)
    out = _combine_call(y, d0, d1, w0x, w1x)
    return out.reshape(bsz, seq, d)


# speedup vs baseline: 1.0959x; 1.0959x over previous
import functools

import jax
import jax.numpy as jnp
from jax import lax
from jax.experimental import pallas as pl
from jax.experimental.pallas import tpu as pltpu
from jax.experimental.pallas import tpu_sc as plsc

T = 2048
D = 1024
DFF = 4096
E = 8
TOPK = 2
P = T * TOPK

TM = 256
NT = (P + E * TM) // TM
M_PAD = NT * TM

NC, NS, LANES = 2, 16, 16
NW = NC * NS

_F32 = jnp.float32
_I32 = jnp.int32


def _cumsum_rows(x):
    n = x.shape[0]
    s = 1
    while s < n:
        x = x + jnp.concatenate(
            [jnp.zeros((s, x.shape[1]), x.dtype), x[:-s, :]], axis=0)
        s *= 2
    return x


def _cumsum_lanes(x):
    n = x.shape[1]
    s = 1
    while s < n:
        x = x + jnp.concatenate(
            [jnp.zeros((x.shape[0], s), x.dtype), x[:, :-s]], axis=1)
        s *= 2
    return x


def _router_body(xf_ref, wr_ref, br_ref, dst_ref, w0_ref, w1_ref, meta_ref):
    xf = xf_ref[...]
    wr = wr_ref[...]
    logits = lax.dot_general(
        xf, wr, (((1,), (0,)), ((), ())),
        preferred_element_type=_F32)
    logits = logits + br_ref[...]
    lane = lax.broadcasted_iota(_I32, (T, 128), 1)
    neg = _F32(-1e30)
    logits = jnp.where(lane < E, logits, neg)

    m0 = jnp.max(logits, axis=1, keepdims=True)
    e0 = jnp.min(jnp.where(logits == m0, lane, 127), axis=1, keepdims=True)
    l2 = jnp.where(lane == e0, neg, logits)
    m1 = jnp.max(l2, axis=1, keepdims=True)
    e1 = jnp.min(jnp.where(l2 == m1, lane, 127), axis=1, keepdims=True)
    r = jnp.exp(m1 - m0)
    w0 = 1.0 / (1.0 + r)
    w1 = 1.0 - w0
    w0_ref[...] = jnp.broadcast_to(w0, (T, 16))
    w1_ref[...] = jnp.broadcast_to(w1, (T, 16))

    onehot = jnp.concatenate([lane == e0, lane == e1], axis=0).astype(_F32)
    csum = _cumsum_rows(onehot)
    pos = jnp.sum((csum - onehot) * onehot, axis=1, keepdims=True)
    counts = csum[P - 1:P, :]
    cap = jnp.ceil(counts / TM) * TM
    poff = _cumsum_lanes(cap) - cap
    dst = jnp.sum(onehot * poff, axis=1, keepdims=True) + pos
    dst_ref[...] = dst.astype(_I32)

    nact = jnp.sum(cap, axis=1, keepdims=True) / TM
    lane_f = lane.astype(_F32)
    tstart = lax.broadcasted_iota(_I32, (32, 128), 0).astype(_F32) * TM
    belongs = (tstart >= poff) & (tstart < poff + cap) & (counts > 0)
    te = jnp.sum(jnp.where(belongs, lane_f[:32, :], 0.0), axis=1,
                 keepdims=True)
    q = jnp.sum(jnp.where(belongs, poff + cap, 0.0), axis=1, keepdims=True)
    b2g = (q >= poff) & (q < poff + cap) & (counts > 0)
    ne_raw = jnp.sum(jnp.where(b2g, lane_f[:32, :], 0.0), axis=1,
                     keepdims=True)
    has_next = jnp.sum(b2g.astype(_F32), axis=1, keepdims=True) > 0
    ne = jnp.where(has_next, ne_raw, te)
    last_e = jnp.max(jnp.where(counts > 0, lane_f[:1, :], -1.0),
                     axis=1, keepdims=True)
    tile_i = lax.broadcasted_iota(_I32, (32, 1), 0).astype(_F32)
    te = jnp.where(tile_i < nact, te, last_e)
    ne = jnp.where(tile_i < nact, ne, last_e)
    meta_ref[...] = jnp.concatenate([nact, te, ne], axis=0).astype(_I32)


def _router_call(xf, wr_p, br_p):
    return pl.pallas_call(
        _router_body,
        out_shape=(
            jax.ShapeDtypeStruct((P, 1), _I32),
            jax.ShapeDtypeStruct((T, 16), _F32),
            jax.ShapeDtypeStruct((T, 16), _F32),
            jax.ShapeDtypeStruct((65, 1), _I32),
        ),
    )(xf, wr_p, br_p)


def _dispatch_call(xf, dst_b):
    mesh = plsc.VectorSubcoreMesh(core_axis_name="c", subcore_axis_name="s")
    ppw = P // NW
    nch = ppw // LANES

    @functools.partial(
        pl.kernel,
        mesh=mesh,
        out_type=jax.ShapeDtypeStruct((M_PAD, D), _F32),
        scratch_types=[
            pltpu.VMEM((nch, LANES), _I32),
            pltpu.VMEM((2, LANES, D), _F32),
            pltpu.SemaphoreType.DMA,
            pltpu.SemaphoreType.DMA,
            pltpu.SemaphoreType.DMA,
            pltpu.SemaphoreType.DMA,
        ],
    )
    def b_kernel(xf_hbm, dst_hbm, xs_hbm, idx_v, rows_v, sl0, sl1, ss0, ss1):
        wid = lax.axis_index("s") * NC + lax.axis_index("c")
        pltpu.sync_copy(dst_hbm.at[wid], idx_v)
        lsem = (sl0, sl1)
        ssem = (ss0, ss1)

        def load_cp(c):
            j0 = wid * ppw + c * LANES
            t0 = lax.rem(j0, T)
            return pltpu.make_async_copy(
                xf_hbm.at[pl.ds(t0, LANES)], rows_v.at[c % 2], lsem[c % 2])

        def scat_cp(c):
            return pltpu.make_async_copy(
                rows_v.at[c % 2], xs_hbm.at[idx_v.at[c]], ssem[c % 2])

        load_cp(0).start()
        for c in range(nch):
            if c >= 1:
                scat_cp(c - 1).wait()
            if c + 1 < nch:
                load_cp(c + 1).start()
            load_cp(c).wait()
            scat_cp(c).start()
        scat_cp(nch - 1).wait()

    return b_kernel(xf, dst_b)


def _ffn_body(meta_ref, xs_ref, b1_ref, b2_ref, w1_hbm, w2_hbm, y_ref,
              w1b, w2b, w1f, w2f, sem1, sem2):
    i = pl.program_id(0)
    nact = meta_ref[0]
    e = meta_ref[1 + i]
    ne = meta_ref[33 + i]
    first = meta_ref[1 + i] != meta_ref[i]

    def w1_copy(eidx):
        return pltpu.make_async_copy(w1_hbm.at[eidx], w1f, sem1)

    def w2_copy(eidx):
        return pltpu.make_async_copy(w2_hbm.at[eidx], w2f, sem2)

    @pl.when(first & (i == 0))
    def _():
        w1_copy(e).start()
        w2_copy(e).start()

    @pl.when(first)
    def _():
        w1_copy(e).wait()
        w2_copy(e).wait()
        w1b[...] = w1f[...].astype(jnp.bfloat16)
        w2b[...] = w2f[...].astype(jnp.bfloat16)

    @pl.when(first & (ne != e))
    def _():
        w1_copy(ne).start()
        w2_copy(ne).start()

    @pl.when(i < nact)
    def _():
        x = xs_ref[...].astype(jnp.bfloat16)
        h = lax.dot_general(
            x, w1b[...], (((1,), (0,)), ((), ())),
            preferred_element_type=_F32)
        h = h + b1_ref[0]
        h = 0.5 * h * (1.0 + lax.erf(h * 0.7071067811865476))
        y = lax.dot_general(
            h.astype(jnp.bfloat16), w2b[...], (((1,), (0,)), ((), ())),
            preferred_element_type=_F32)
        y_ref[...] = y + b2_ref[0]


def _ffn_call(meta, xs, b1r, b2r, w1, w2):
    grid_spec = pltpu.PrefetchScalarGridSpec(
        num_scalar_prefetch=1,
        grid=(NT,),
        in_specs=[
            pl.BlockSpec((TM, D), lambda i, m: (i, 0)),
            pl.BlockSpec((1, 1, DFF), lambda i, m: (m[1 + i], 0, 0)),
            pl.BlockSpec((1, 1, D), lambda i, m: (m[1 + i], 0, 0)),
            pl.BlockSpec(memory_space=pl.ANY),
            pl.BlockSpec(memory_space=pl.ANY),
        ],
        out_specs=pl.BlockSpec((TM, D), lambda i, m: (i, 0)),
        scratch_shapes=[
            pltpu.VMEM((D, DFF), jnp.bfloat16),
            pltpu.VMEM((DFF, D), jnp.bfloat16),
            pltpu.VMEM((D, DFF), _F32),
            pltpu.VMEM((DFF, D), _F32),
            pltpu.SemaphoreType.DMA,
            pltpu.SemaphoreType.DMA,
        ],
    )
    return pl.pallas_call(
        _ffn_body,
        grid_spec=grid_spec,
        out_shape=jax.ShapeDtypeStruct((M_PAD, D), _F32),
    )(meta, xs, b1r, b2r, w1, w2)


def _combine_call(y, d0, d1, w0x, w1x):
    mesh = plsc.VectorSubcoreMesh(core_axis_name="c", subcore_axis_name="s")
    tpw = T // NW
    nch = tpw // LANES

    @functools.partial(
        pl.kernel,
        mesh=mesh,
        out_type=jax.ShapeDtypeStruct((T, D), _F32),
        scratch_types=[
            pltpu.VMEM((nch, LANES), _I32),
            pltpu.VMEM((nch, LANES), _I32),
            pltpu.VMEM((tpw, LANES), _F32),
            pltpu.VMEM((tpw, LANES), _F32),
            pltpu.VMEM((2, LANES, D), _F32),
            pltpu.VMEM((2, LANES, D), _F32),
            pltpu.VMEM((2, LANES, D), _F32),
            pltpu.SemaphoreType.DMA,
            pltpu.SemaphoreType.DMA,
            pltpu.SemaphoreType.DMA,
            pltpu.SemaphoreType.DMA,
        ],
    )
    def d_kernel(y_hbm, d0_hbm, d1_hbm, w0_hbm, w1_hbm, out_hbm,
                 i0, i1, w0v, w1v, y0, y1, ov, sg0, sg1, sw0, sw1):
        wid = lax.axis_index("s") * NC + lax.axis_index("c")
        pltpu.sync_copy(d0_hbm.at[wid], i0)
        pltpu.sync_copy(d1_hbm.at[wid], i1)
        pltpu.sync_copy(w0_hbm.at[pl.ds(wid * tpw, tpw)], w0v)
        pltpu.sync_copy(w1_hbm.at[pl.ds(wid * tpw, tpw)], w1v)
        gsem = (sg0, sg1)
        wsem = (sw0, sw1)

        def g0_cp(c):
            return pltpu.make_async_copy(
                y_hbm.at[i0.at[c]], y0.at[c % 2], gsem[c % 2])

        def g1_cp(c):
            return pltpu.make_async_copy(
                y_hbm.at[i1.at[c]], y1.at[c % 2], gsem[c % 2])

        def out_cp(c):
            tbase = wid * tpw + c * LANES
            return pltpu.make_async_copy(
                ov.at[c % 2], out_hbm.at[pl.ds(tbase, LANES)], wsem[c % 2])

        g0_cp(0).start()
        g1_cp(0).start()
        for c in range(nch):
            if c + 1 < nch:
                g0_cp(c + 1).start()
                g1_cp(c + 1).start()
            g0_cp(c).wait()
            g1_cp(c).wait()
            if c >= 2:
                out_cp(c - 2).wait()
            par = c % 2

            def row(rr, rcarry, c=c, par=par):
                a0 = w0v[c * LANES + rr]
                a1 = w1v[c * LANES + rr]
                for cc in range(D // LANES):
                    sl = pl.ds(cc * LANES, LANES)
                    ov[par, rr, sl] = (a0 * y0[par, rr, sl]
                                       + a1 * y1[par, rr, sl])
                return rcarry

            lax.fori_loop(0, LANES, row, 0)
            out_cp(c).start()
        out_cp(nch - 2).wait()
        out_cp(nch - 1).wait()

    return d_kernel(y, d0, d1, w0x, w1x)


def kernel(x, W1, b1, W2, b2, Wr, br):
    bsz, seq, d = x.shape
    xf = x.reshape(T, D)
    wr_p = jnp.pad(Wr, ((0, 0), (0, 128 - E)))
    br_p = jnp.pad(br, (0, 128 - E)).reshape(1, 128)

    dst, w0x, w1x, meta = _router_call(xf, wr_p, br_p)
    dst_flat = dst.reshape(P)
    dst_b = dst_flat.reshape(NW, P // NW // LANES, LANES)

    xs = _dispatch_call(xf, dst_b)

    y = _ffn_call(meta.reshape(65), xs, b1.reshape(E, 1, DFF),
                  b2.reshape(E, 1, D), W1, W2)

    d0 = dst_flat[:T].reshape(NW, T // NW // LANES, LANES)
    d1 = dst_flat[T:].reshape(NW, T // NW // LANES, LANES)
    out = _combine_call(y, d0, d1, w0x, w1x)
    return out.reshape(bsz, seq, d)
